# Initial kernel scaffold; baseline (speedup 1.0000x reference)
import jax
import jax.numpy as jnp
from jax.experimental import pallas as pl


def kernel(x, edge_index, W1, b1, W2, b2):
    raise NotImplementedError("write your pallas kernel here")


# speedup vs baseline: 9.7488x; 9.7488x over previous
import functools

import jax
import jax.numpy as jnp
from jax import lax
from jax.experimental import pallas as pl
from jax.experimental.pallas import tpu as pltpu
from jax.experimental.pallas import tpu_sc as plsc

MU_C = 0.1
NC = 2
NS = 16
CHUNK = 128


def _fill_f32(ref, rows, width, value):
    vals = jnp.full((16,), value, jnp.float32)

    def body(i, carry):
        for t in range(width // 16):
            ref[i, pl.ds(t * 16, 16)] = vals
        return carry

    lax.fori_loop(0, rows, body, 0)


def _make_sc_pass(npad, w, nchunks, gather):
    mesh = plsc.VectorSubcoreMesh(core_axis_name="c", subcore_axis_name="s")
    cpt = nchunks // (NC * NS)
    rps = npad // NS
    ncopy = rps // CHUNK

    scratch = [
        pltpu.VMEM((CHUNK, w), jnp.float32),
        pltpu.VMEM((CHUNK,), jnp.int32),
        pltpu.VMEM((CHUNK,), jnp.int32),
        pltpu.VMEM_SHARED((npad, w), jnp.float32),
        pltpu.SemaphoreType.DMA,
    ]
    out_type = jax.ShapeDtypeStruct((NC, npad, w), jnp.float32)

    @functools.partial(
        pl.kernel, out_type=out_type, mesh=mesh, scratch_types=scratch,
        compiler_params=pltpu.CompilerParams(use_tc_tiling_on_sc=False))
    def sc_pass(t_hbm, col_hbm, row_hbm, out_hbm, rows_v, ridx_v, cidx_v,
                acc, sem):
        c = lax.axis_index("c")
        s = lax.axis_index("s")
        tid = c * NS + s

        _fill_f32(rows_v, CHUNK, w, 0.0)
        for k in range(ncopy):
            pltpu.sync_copy(rows_v,
                            acc.at[pl.ds(s * rps + k * CHUNK, CHUNK)])
        if not gather:
            _fill_f32(rows_v, CHUNK, w, 1.0)
        plsc.subcore_barrier()

        def body(j, carry):
            ch = tid * cpt + j
            pltpu.sync_copy(row_hbm.at[ch], ridx_v)
            if gather:
                pltpu.sync_copy(col_hbm.at[ch], cidx_v)
                pltpu.async_copy(t_hbm.at[cidx_v], rows_v, sem).wait()
            pltpu.sync_copy(rows_v, acc.at[ridx_v], add=True)
            return carry

        lax.fori_loop(0, cpt, body, 0)
        plsc.subcore_barrier()
        pltpu.sync_copy(acc.at[pl.ds(s * rps, rps)],
                        out_hbm.at[c, pl.ds(s * rps, rps)])

    return sc_pass


def _tc_prep(x, w1, b1r, degp, blk):
    n, din = x.shape
    dh = w1.shape[0]
    wt = dh + 16
    g = n // blk

    def body(x_ref, w1_ref, b1_ref, degp_ref, h_ref, t1_ref, dinv_ref):
        xx = x_ref[...]
        h = lax.dot_general(xx, w1_ref[...], (((1,), (1,)), ((), ())),
                            precision=lax.Precision.HIGHEST)
        h = jnp.maximum(h + b1_ref[...], 0.0)
        deg = degp_ref[0, :, 0] + degp_ref[1, :, 0] + 1.0
        dinv = lax.rsqrt(deg)[:, None]
        h_ref[...] = h
        dinv_ref[...] = dinv
        t1_ref[...] = jnp.concatenate(
            [dinv * h, dinv, jnp.zeros((blk, 15), jnp.float32)], axis=1)

    return pl.pallas_call(
        body,
        grid=(g,),
        in_specs=[
            pl.BlockSpec((blk, din), lambda i: (i, 0)),
            pl.BlockSpec((dh, din), lambda i: (0, 0)),
            pl.BlockSpec((1, dh), lambda i: (0, 0)),
            pl.BlockSpec((2, blk, 16), lambda i: (0, i, 0)),
        ],
        out_specs=[
            pl.BlockSpec((blk, dh), lambda i: (i, 0)),
            pl.BlockSpec((blk, wt), lambda i: (i, 0)),
            pl.BlockSpec((blk, 1), lambda i: (i, 0)),
        ],
        out_shape=[
            jax.ShapeDtypeStruct((n, dh), jnp.float32),
            jax.ShapeDtypeStruct((n, wt), jnp.float32),
            jax.ShapeDtypeStruct((n, 1), jnp.float32),
        ],
    )(x, w1, b1r, degp)


def _tc_combine1(parts, dinv, h, blk):
    n, dh = h.shape
    wt = parts.shape[2]
    g = n // blk

    def body(p_ref, dinv_ref, h_ref, out1_ref, t2_ref, ab_ref):
        p = p_ref[0] + p_ref[1]
        s128 = p[:, :dh]
        scol = p[:, dh:dh + 1]
        dv = dinv_ref[...]
        deg_m = dv * scol + dv * dv
        alpha = 1.0 / (MU_C + deg_m)
        beta = MU_C * alpha
        hh = h_ref[...]
        out1 = alpha * (dv * s128 + dv * dv * hh) + beta * hh
        out1_ref[...] = out1
        t2_ref[...] = dv * out1
        ab_ref[...] = jnp.concatenate([alpha, beta], axis=1)

    return pl.pallas_call(
        body,
        grid=(g,),
        in_specs=[
            pl.BlockSpec((2, blk, wt), lambda i: (0, i, 0)),
            pl.BlockSpec((blk, 1), lambda i: (i, 0)),
            pl.BlockSpec((blk, dh), lambda i: (i, 0)),
        ],
        out_specs=[
            pl.BlockSpec((blk, dh), lambda i: (i, 0)),
            pl.BlockSpec((blk, dh), lambda i: (i, 0)),
            pl.BlockSpec((blk, 2), lambda i: (i, 0)),
        ],
        out_shape=[
            jax.ShapeDtypeStruct((n, dh), jnp.float32),
            jax.ShapeDtypeStruct((n, dh), jnp.float32),
            jax.ShapeDtypeStruct((n, 2), jnp.float32),
        ],
    )(parts, dinv, h)


def _tc_combine2(parts, dinv, out1, h, ab, w2, b2r, blk):
    n, dh = h.shape
    dout = w2.shape[0]
    g = n // blk

    def body(p_ref, dinv_ref, out1_ref, h_ref, ab_ref, w2_ref, b2_ref, y_ref):
        p = p_ref[0] + p_ref[1]
        dv = dinv_ref[...]
        alpha = ab_ref[:, 0:1]
        beta = ab_ref[:, 1:2]
        o1 = out1_ref[...]
        out2 = alpha * (dv * p + dv * dv * o1) + beta * h_ref[...]
        y = lax.dot_general(out2, w2_ref[...], (((1,), (1,)), ((), ())),
                            precision=lax.Precision.HIGHEST) + b2_ref[...]
        m = jnp.max(y, axis=1, keepdims=True)
        lse = m + jnp.log(jnp.sum(jnp.exp(y - m), axis=1, keepdims=True))
        y_ref[...] = y - lse

    return pl.pallas_call(
        body,
        grid=(g,),
        in_specs=[
            pl.BlockSpec((2, blk, dh), lambda i: (0, i, 0)),
            pl.BlockSpec((blk, 1), lambda i: (i, 0)),
            pl.BlockSpec((blk, dh), lambda i: (i, 0)),
            pl.BlockSpec((blk, dh), lambda i: (i, 0)),
            pl.BlockSpec((blk, 2), lambda i: (i, 0)),
            pl.BlockSpec((dout, dh), lambda i: (0, 0)),
            pl.BlockSpec((1, dout), lambda i: (0, 0)),
        ],
        out_specs=pl.BlockSpec((blk, dout), lambda i: (i, 0)),
        out_shape=jax.ShapeDtypeStruct((n, dout), jnp.float32),
    )(parts, dinv, out1, h, ab, w2, b2r)


def kernel(x, edge_index, W1, b1, W2, b2):
    n, _ = x.shape
    dh = W1.shape[0]
    e = edge_index.shape[1]
    blk = 1000 if n % 1000 == 0 else 8

    npad = -(-(n + 1) // (NS * CHUNK)) * (NS * CHUNK)
    nchunks = -(-e // CHUNK)
    nchunks = -(-nchunks // (NC * NS)) * (NC * NS)
    epad = nchunks * CHUNK - e

    row = edge_index[0].astype(jnp.int32)
    col = edge_index[1].astype(jnp.int32)
    padv = jnp.full((epad,), n, jnp.int32)
    rowc = jnp.concatenate([row, padv]).reshape(nchunks, CHUNK)
    colc = jnp.concatenate([col, padv]).reshape(nchunks, CHUNK)

    deg_pass = _make_sc_pass(npad, 16, nchunks, gather=False)
    dummy_t = jnp.zeros((8, 16), jnp.float32)
    degp = deg_pass(dummy_t, rowc, rowc)[:, :n, :]

    h, t1c, dinv = _tc_prep(x, W1, b1.reshape(1, -1), degp, blk)
    t1 = jnp.pad(t1c, ((0, npad - n), (0, 0)))

    spmm_a = _make_sc_pass(npad, dh + 16, nchunks, gather=True)
    p1 = spmm_a(t1, colc, rowc)[:, :n, :]

    out1, t2c, ab = _tc_combine1(p1, dinv, h, blk)
    t2 = jnp.pad(t2c, ((0, npad - n), (0, 0)))

    spmm_b = _make_sc_pass(npad, dh, nchunks, gather=True)
    p2 = spmm_b(t2, colc, rowc)[:, :n, :]

    return _tc_combine2(p2, dinv, out1, h, ab, W2, b2.reshape(1, -1), blk)
